# baseline (speedup 1.0000x reference)
import functools

import jax
import jax.numpy as jnp
from jax import lax
from jax.experimental import pallas as pl
from jax.experimental.pallas import tpu as pltpu
from jax.experimental.pallas import tpu_sc as plsc

_N_CIR = 585
_N_DIS = 88
_D = 128
_E_CC = 11700
_E_DD = 1760

_N_CIR_P = 640
_N_DIS_P = 128
_E_CC_P = 12288
_E_DD_P = 2048

_NC = 2
_NS = 16
_NW = _NC * _NS
_CC_ROWS = _N_CIR_P // _NW
_DD_ROWS = _N_DIS_P // _NW


def _sc_count_matrices(ecc, edd, zeros):
    mesh = plsc.VectorSubcoreMesh(core_axis_name="c", subcore_axis_name="s")

    @functools.partial(
        pl.kernel,
        out_type=(
            jax.ShapeDtypeStruct((_N_CIR_P * _N_CIR_P,), jnp.float32),
            jax.ShapeDtypeStruct((_N_DIS_P * _N_DIS_P,), jnp.float32),
        ),
        mesh=mesh,
        compiler_params=pltpu.CompilerParams(needs_layout_passes=False),
        scratch_types=[
            pltpu.VMEM((2, _E_CC_P), jnp.int32),
            pltpu.VMEM((2, _E_DD_P), jnp.int32),
            pltpu.VMEM((_CC_ROWS * _N_CIR_P,), jnp.float32),
            pltpu.VMEM((_DD_ROWS * _N_DIS_P,), jnp.float32),
        ],
    )
    def k(ecc_hbm, edd_hbm, zeros_hbm, outc_hbm, outd_hbm,
          ecc_v, edd_v, cntc_v, cntd_v):
        wid = lax.axis_index("s") * _NC + lax.axis_index("c")
        pltpu.sync_copy(ecc_hbm, ecc_v)
        pltpu.sync_copy(edd_hbm, edd_v)
        pltpu.sync_copy(zeros_hbm, cntc_v)
        pltpu.sync_copy(zeros_hbm.at[pl.ds(0, _DD_ROWS * _N_DIS_P)], cntd_v)

        lane = lax.iota(jnp.int32, 16)
        ones = jnp.full((16,), 1.0, jnp.float32)

        def edge_scan(ev, cnt_v, n_vec, lo, hi, npad):
            def body(i, carry):
                base = i * 16
                r = ev[0, pl.ds(base, 16)]
                c = ev[1, pl.ds(base, 16)]
                m = (c >= lo) & (c < hi)
                li = (c - lo) * npad + r
                for j in range(16):
                    plsc.addupdate_scatter(
                        cnt_v, [li], ones, mask=m & (lane == j))
                return carry

            lax.fori_loop(0, n_vec, body, 0)

        lo_c = wid * _CC_ROWS
        edge_scan(ecc_v, cntc_v, _E_CC_P // 16, lo_c, lo_c + _CC_ROWS, _N_CIR_P)
        lo_d = wid * _DD_ROWS
        edge_scan(edd_v, cntd_v, _E_DD_P // 16, lo_d, lo_d + _DD_ROWS, _N_DIS_P)

        pltpu.sync_copy(
            cntc_v,
            outc_hbm.at[pl.ds(wid * _CC_ROWS * _N_CIR_P, _CC_ROWS * _N_CIR_P)])
        pltpu.sync_copy(
            cntd_v,
            outd_hbm.at[pl.ds(wid * _DD_ROWS * _N_DIS_P, _DD_ROWS * _N_DIS_P)])

    return k(ecc, edd, zeros)


def _tc_dense_body(cntc_ref, mct_ref, xc_ref, wc1_ref, bc1_ref, wc2_ref,
                   bc2_ref, uc0_ref, uc1_ref, bcc_ref,
                   cntd_ref, mdt_ref, xd_ref, wd1_ref, bd1_ref, wd2_ref,
                   bd2_ref, ud0_ref, ud1_ref, bdc_ref,
                   score_ref, cir_ref, dis_ref):
    f32 = jnp.float32

    def dot(a, b):
        return jnp.dot(a, b, preferred_element_type=f32,
                       precision=lax.Precision.HIGHEST)

    def side(cnt, mt, x, w1, b1, w2, b2, u0, u1, bc, n):
        B = cnt * mt
        ones = jnp.ones((n, 1), f32)
        deg = 1.0 + dot(B, ones)
        dinv = lax.rsqrt(deg)

        def gcn(xin, W, b):
            h = dot(xin, W)
            t = dinv * h
            u = dot(B, t)
            return jnp.maximum(dinv * u + (dinv * dinv) * h + b, 0.0)

        f1 = gcn(x, w1, b1)
        f2 = gcn(f1, w2, b2)
        return dot(f1, u0) + dot(f2, u1) + bc

    cir = side(cntc_ref[...], mct_ref[...], xc_ref[...], wc1_ref[...],
               bc1_ref[...], wc2_ref[...], bc2_ref[...], uc0_ref[...],
               uc1_ref[...], bcc_ref[...], _N_CIR_P)
    dis = side(cntd_ref[...], mdt_ref[...], xd_ref[...], wd1_ref[...],
               bd1_ref[...], wd2_ref[...], bd2_ref[...], ud0_ref[...],
               ud1_ref[...], bdc_ref[...], _N_DIS_P)
    score_ref[...] = lax.dot_general(
        cir, dis, (((1,), (1,)), ((), ())), preferred_element_type=f32,
        precision=lax.Precision.HIGHEST)
    cir_ref[...] = cir
    dis_ref[...] = dis


def kernel(x_cir, x_dis, cc_matrix, cc_edges, dd_matrix, dd_edges,
           W_cir1, b_cir1, W_cir2, b_cir2, W_dis1, b_dis1, W_dis2, b_dis2,
           W_cnn_cir, b_cnn_cir, W_cnn_dis, b_cnn_dis):
    f32 = jnp.float32

    pad_cc = jnp.broadcast_to(
        jnp.array([[0], [_N_CIR_P - 1]], jnp.int32), (2, _E_CC_P - _E_CC))
    pad_dd = jnp.broadcast_to(
        jnp.array([[0], [_N_DIS_P - 1]], jnp.int32), (2, _E_DD_P - _E_DD))
    ecc = jnp.concatenate([cc_edges.astype(jnp.int32), pad_cc], axis=1)
    edd = jnp.concatenate([dd_edges.astype(jnp.int32), pad_dd], axis=1)
    zeros = jnp.zeros((_CC_ROWS * _N_CIR_P,), f32)

    cntc_flat, cntd_flat = _sc_count_matrices(ecc, edd, zeros)
    cntc = cntc_flat.reshape(_N_CIR_P, _N_CIR_P)
    cntd = cntd_flat.reshape(_N_DIS_P, _N_DIS_P)

    pc = _N_CIR_P - _N_CIR
    pd = _N_DIS_P - _N_DIS
    mct = jnp.pad(cc_matrix.T, ((0, pc), (0, pc)))
    mdt = jnp.pad(dd_matrix.T, ((0, pd), (0, pd)))
    xc = jnp.pad(x_cir, ((0, pc), (0, 0)))
    xd = jnp.pad(x_dis, ((0, pd), (0, 0)))
    uc0 = W_cnn_cir[:, 0, :].T
    uc1 = W_cnn_cir[:, 1, :].T
    ud0 = W_cnn_dis[:, 0, :].T
    ud1 = W_cnn_dis[:, 1, :].T

    score, cir, dis = pl.pallas_call(
        _tc_dense_body,
        out_shape=(
            jax.ShapeDtypeStruct((_N_CIR_P, _N_DIS_P), f32),
            jax.ShapeDtypeStruct((_N_CIR_P, 256), f32),
            jax.ShapeDtypeStruct((_N_DIS_P, 256), f32),
        ),
    )(cntc, mct, xc, W_cir1, b_cir1.reshape(1, _D), W_cir2,
      b_cir2.reshape(1, _D), uc0, uc1, b_cnn_cir.reshape(1, 256),
      cntd, mdt, xd, W_dis1, b_dis1.reshape(1, _D), W_dis2,
      b_dis2.reshape(1, _D), ud0, ud1, b_cnn_dis.reshape(1, 256))

    return (score[:_N_CIR, :_N_DIS], cir[:_N_CIR], dis[:_N_DIS])


# speedup vs baseline: 1.4861x; 1.4861x over previous
import functools

import jax
import jax.numpy as jnp
from jax import lax
from jax.experimental import pallas as pl
from jax.experimental.pallas import tpu as pltpu
from jax.experimental.pallas import tpu_sc as plsc

_N_CIR = 585
_N_DIS = 88
_D = 128
_E_CC = 11700
_E_DD = 1760

_N_CIR_P = 640
_N_DIS_P = 128
_E_CC_P = 12288
_E_DD_P = 2048

_NC = 2
_NS = 16
_NW = _NC * _NS
_CC_ROWS = _N_CIR_P // _NW
_DD_ROWS = _N_DIS_P // _NW


_CC_PER_TILE = _E_CC_P // _NS
_DD_PER_TILE = _E_DD_P // _NS
_CC_STRIPE = _N_CIR_P * _N_CIR_P // _NS
_DD_STRIPE = _N_DIS_P * _N_DIS_P // _NS


def _sc_count_matrices(ecc, edd, zeros):
    mesh = plsc.VectorSubcoreMesh(core_axis_name="c", subcore_axis_name="s")

    @functools.partial(
        pl.kernel,
        out_type=(
            jax.ShapeDtypeStruct((_N_CIR_P * _N_CIR_P,), jnp.float32),
            jax.ShapeDtypeStruct((_N_DIS_P * _N_DIS_P,), jnp.float32),
        ),
        mesh=mesh,
        compiler_params=pltpu.CompilerParams(needs_layout_passes=False),
        scratch_types=[
            pltpu.VMEM((_CC_PER_TILE,), jnp.int32),
            pltpu.VMEM((_CC_PER_TILE,), jnp.int32),
            pltpu.VMEM((_CC_PER_TILE // 128, 128), jnp.int32),
            pltpu.VMEM((128,), jnp.float32),
            pltpu.VMEM_SHARED((_N_CIR_P * _N_CIR_P,), jnp.float32),
        ],
    )
    def k(ecc_hbm, edd_hbm, zeros_hbm, outc_hbm, outd_hbm,
          er_v, ec_v, idx_v, ones_v, shr):
        core = lax.axis_index("c")
        s = lax.axis_index("s")
        for q in range(8):
            ones_v[pl.ds(q * 16, 16)] = jnp.full((16,), 1.0, jnp.float32)

        def graph(e_hbm, out_hbm, per_tile, stripe, npad):
            n_chunk = per_tile // 128
            base = s * per_tile
            pltpu.sync_copy(e_hbm.at[0, pl.ds(base, per_tile)],
                            er_v.at[pl.ds(0, per_tile)])
            pltpu.sync_copy(e_hbm.at[1, pl.ds(base, per_tile)],
                            ec_v.at[pl.ds(0, per_tile)])
            for q in range(per_tile // 16):
                r = er_v[pl.ds(q * 16, 16)]
                c = ec_v[pl.ds(q * 16, 16)]
                idx_v[q // 8, pl.ds((q % 8) * 16, 16)] = c * npad + r
            pltpu.sync_copy(zeros_hbm.at[pl.ds(0, stripe)],
                            shr.at[pl.ds(s * stripe, stripe)])
            plsc.subcore_barrier()
            for j in range(n_chunk):
                pltpu.sync_copy(ones_v, shr.at[idx_v.at[j]], add=True)
            plsc.subcore_barrier()
            pltpu.sync_copy(shr.at[pl.ds(s * stripe, stripe)],
                            out_hbm.at[pl.ds(s * stripe, stripe)])

        @pl.when(core == 0)
        def _():
            graph(ecc_hbm, outc_hbm, _CC_PER_TILE, _CC_STRIPE, _N_CIR_P)

        @pl.when(core == 1)
        def _():
            graph(edd_hbm, outd_hbm, _DD_PER_TILE, _DD_STRIPE, _N_DIS_P)

    return k(ecc, edd, zeros)


def _tc_dense_body(cntc_ref, mct_ref, xc_ref, wc1_ref, bc1_ref, wc2_ref,
                   bc2_ref, uc0_ref, uc1_ref, bcc_ref,
                   cntd_ref, mdt_ref, xd_ref, wd1_ref, bd1_ref, wd2_ref,
                   bd2_ref, ud0_ref, ud1_ref, bdc_ref,
                   score_ref, cir_ref, dis_ref):
    f32 = jnp.float32

    def dot(a, b):
        return jnp.dot(a, b, preferred_element_type=f32,
                       precision=lax.Precision.HIGHEST)

    def side(cnt, mt, x, w1, b1, w2, b2, u0, u1, bc, n):
        B = cnt * mt
        ones = jnp.ones((n, 1), f32)
        deg = 1.0 + dot(B, ones)
        dinv = lax.rsqrt(deg)

        def gcn(xin, W, b):
            h = dot(xin, W)
            t = dinv * h
            u = dot(B, t)
            return jnp.maximum(dinv * u + (dinv * dinv) * h + b, 0.0)

        f1 = gcn(x, w1, b1)
        f2 = gcn(f1, w2, b2)
        return dot(f1, u0) + dot(f2, u1) + bc

    cir = side(cntc_ref[...], mct_ref[...], xc_ref[...], wc1_ref[...],
               bc1_ref[...], wc2_ref[...], bc2_ref[...], uc0_ref[...],
               uc1_ref[...], bcc_ref[...], _N_CIR_P)
    dis = side(cntd_ref[...], mdt_ref[...], xd_ref[...], wd1_ref[...],
               bd1_ref[...], wd2_ref[...], bd2_ref[...], ud0_ref[...],
               ud1_ref[...], bdc_ref[...], _N_DIS_P)
    score_ref[...] = lax.dot_general(
        cir, dis, (((1,), (1,)), ((), ())), preferred_element_type=f32,
        precision=lax.Precision.HIGHEST)
    cir_ref[...] = cir
    dis_ref[...] = dis


def kernel(x_cir, x_dis, cc_matrix, cc_edges, dd_matrix, dd_edges,
           W_cir1, b_cir1, W_cir2, b_cir2, W_dis1, b_dis1, W_dis2, b_dis2,
           W_cnn_cir, b_cnn_cir, W_cnn_dis, b_cnn_dis):
    f32 = jnp.float32

    pad_cc = jnp.broadcast_to(
        jnp.array([[0], [_N_CIR_P - 1]], jnp.int32), (2, _E_CC_P - _E_CC))
    pad_dd = jnp.broadcast_to(
        jnp.array([[0], [_N_DIS_P - 1]], jnp.int32), (2, _E_DD_P - _E_DD))
    ecc = jnp.concatenate([cc_edges.astype(jnp.int32), pad_cc], axis=1)
    edd = jnp.concatenate([dd_edges.astype(jnp.int32), pad_dd], axis=1)
    zeros = jnp.zeros((_CC_STRIPE,), f32)

    cntc_flat, cntd_flat = _sc_count_matrices(ecc, edd, zeros)
    cntc = cntc_flat.reshape(_N_CIR_P, _N_CIR_P)
    cntd = cntd_flat.reshape(_N_DIS_P, _N_DIS_P)

    pc = _N_CIR_P - _N_CIR
    pd = _N_DIS_P - _N_DIS
    mct = jnp.pad(cc_matrix.T, ((0, pc), (0, pc)))
    mdt = jnp.pad(dd_matrix.T, ((0, pd), (0, pd)))
    xc = jnp.pad(x_cir, ((0, pc), (0, 0)))
    xd = jnp.pad(x_dis, ((0, pd), (0, 0)))
    uc0 = W_cnn_cir[:, 0, :].T
    uc1 = W_cnn_cir[:, 1, :].T
    ud0 = W_cnn_dis[:, 0, :].T
    ud1 = W_cnn_dis[:, 1, :].T

    score, cir, dis = pl.pallas_call(
        _tc_dense_body,
        out_shape=(
            jax.ShapeDtypeStruct((_N_CIR_P, _N_DIS_P), f32),
            jax.ShapeDtypeStruct((_N_CIR_P, 256), f32),
            jax.ShapeDtypeStruct((_N_DIS_P, 256), f32),
        ),
    )(cntc, mct, xc, W_cir1, b_cir1.reshape(1, _D), W_cir2,
      b_cir2.reshape(1, _D), uc0, uc1, b_cnn_cir.reshape(1, 256),
      cntd, mdt, xd, W_dis1, b_dis1.reshape(1, _D), W_dis2,
      b_dis2.reshape(1, _D), ud0, ud1, b_cnn_dis.reshape(1, 256))

    return (score[:_N_CIR, :_N_DIS], cir[:_N_CIR], dis[:_N_DIS])


# speedup vs baseline: 1.5445x; 1.0393x over previous
import functools

import jax
import jax.numpy as jnp
from jax import lax
from jax.experimental import pallas as pl
from jax.experimental.pallas import tpu as pltpu
from jax.experimental.pallas import tpu_sc as plsc

_N_CIR = 585
_N_DIS = 88
_D = 128
_E_CC = 11700
_E_DD = 1760

_N_CIR_P = 640
_N_DIS_P = 128
_E_CC_P = 12288
_E_DD_P = 2048

_NC = 2
_NS = 16
_NW = _NC * _NS
_CC_ROWS = _N_CIR_P // _NW
_DD_ROWS = _N_DIS_P // _NW


_CC_PER_TILE = _E_CC_P // _NS
_DD_PER_TILE = _E_DD_P // _NS
_CC_STRIPE = _N_CIR_P * _N_CIR_P // _NS
_DD_STRIPE = _N_DIS_P * _N_DIS_P // _NS


def _sc_count_matrices(ecc, edd, zeros):
    mesh = plsc.VectorSubcoreMesh(core_axis_name="c", subcore_axis_name="s")

    @functools.partial(
        pl.kernel,
        out_type=(
            jax.ShapeDtypeStruct((_N_CIR_P * _N_CIR_P,), jnp.float32),
            jax.ShapeDtypeStruct((_N_DIS_P * _N_DIS_P,), jnp.float32),
        ),
        mesh=mesh,
        compiler_params=pltpu.CompilerParams(needs_layout_passes=False),
        scratch_types=[
            pltpu.VMEM((_CC_PER_TILE,), jnp.int32),
            pltpu.VMEM((_CC_PER_TILE,), jnp.int32),
            pltpu.VMEM((_CC_PER_TILE // 128, 128), jnp.int32),
            pltpu.VMEM((128,), jnp.float32),
            pltpu.VMEM_SHARED((_N_CIR_P * _N_CIR_P,), jnp.float32),
        ],
    )
    def k(ecc_hbm, edd_hbm, zeros_hbm, outc_hbm, outd_hbm,
          er_v, ec_v, idx_v, ones_v, shr):
        core = lax.axis_index("c")
        s = lax.axis_index("s")
        for q in range(8):
            ones_v[pl.ds(q * 16, 16)] = jnp.full((16,), 1.0, jnp.float32)

        def graph(e_hbm, out_hbm, per_tile, stripe, npad):
            n_chunk = per_tile // 128
            base = s * per_tile
            pltpu.sync_copy(e_hbm.at[0, pl.ds(base, per_tile)],
                            er_v.at[pl.ds(0, per_tile)])
            pltpu.sync_copy(e_hbm.at[1, pl.ds(base, per_tile)],
                            ec_v.at[pl.ds(0, per_tile)])
            for q in range(per_tile // 16):
                r = er_v[pl.ds(q * 16, 16)]
                c = ec_v[pl.ds(q * 16, 16)]
                idx_v[q // 8, pl.ds((q % 8) * 16, 16)] = r * npad + c
            pltpu.sync_copy(zeros_hbm.at[pl.ds(0, stripe)],
                            shr.at[pl.ds(s * stripe, stripe)])
            plsc.subcore_barrier()
            for j in range(n_chunk):
                pltpu.sync_copy(ones_v, shr.at[idx_v.at[j]], add=True)
            plsc.subcore_barrier()
            pltpu.sync_copy(shr.at[pl.ds(s * stripe, stripe)],
                            out_hbm.at[pl.ds(s * stripe, stripe)])

        @pl.when(core == 0)
        def _():
            graph(ecc_hbm, outc_hbm, _CC_PER_TILE, _CC_STRIPE, _N_CIR_P)

        @pl.when(core == 1)
        def _():
            graph(edd_hbm, outd_hbm, _DD_PER_TILE, _DD_STRIPE, _N_DIS_P)

    return k(ecc, edd, zeros)


def _tc_dense_body(cntc_ref, mc_ref, xc_ref, wc1_ref, bc1_ref, wc2_ref,
                   bc2_ref, wcnnc_ref, bcc_ref,
                   cntd_ref, md_ref, xd_ref, wd1_ref, bd1_ref, wd2_ref,
                   bd2_ref, wcnnd_ref, bdc_ref,
                   score_ref, cir_ref, dis_ref):
    f32 = jnp.float32

    def dg(a, b, dims):
        return lax.dot_general(a, b, (dims, ((), ())),
                               preferred_element_type=f32,
                               precision=lax.Precision.HIGHEST)

    def side(cnt, m, x, w1, b1, w2, b2, wcnn, bc, n):
        B = cnt * m
        ones = jnp.ones((n, 1), f32)
        deg = 1.0 + dg(B, ones, ((0,), (0,)))
        dinv = lax.rsqrt(deg)

        def gcn(xin, W, b):
            h = dg(xin, W, ((1,), (0,)))
            t = dinv * h
            u = dg(B, t, ((0,), (0,)))
            return jnp.maximum(dinv * u + (dinv * dinv) * h + b, 0.0)

        f1 = gcn(x, w1, b1)
        f2 = gcn(f1, w2, b2)
        return (dg(f1, wcnn[:, 0, :], ((1,), (1,)))
                + dg(f2, wcnn[:, 1, :], ((1,), (1,))) + bc)

    cir = side(cntc_ref[...], mc_ref[...], xc_ref[...], wc1_ref[...],
               bc1_ref[...], wc2_ref[...], bc2_ref[...], wcnnc_ref[...],
               bcc_ref[...], _N_CIR_P)
    dis = side(cntd_ref[...], md_ref[...], xd_ref[...], wd1_ref[...],
               bd1_ref[...], wd2_ref[...], bd2_ref[...], wcnnd_ref[...],
               bdc_ref[...], _N_DIS_P)
    score_ref[...] = dg(cir, dis, ((1,), (1,)))[:_N_CIR, :_N_DIS]
    cir_ref[...] = cir[:_N_CIR]
    dis_ref[...] = dis[:_N_DIS]


def kernel(x_cir, x_dis, cc_matrix, cc_edges, dd_matrix, dd_edges,
           W_cir1, b_cir1, W_cir2, b_cir2, W_dis1, b_dis1, W_dis2, b_dis2,
           W_cnn_cir, b_cnn_cir, W_cnn_dis, b_cnn_dis):
    f32 = jnp.float32

    pad_cc = jnp.broadcast_to(
        jnp.array([[0], [_N_CIR_P - 1]], jnp.int32), (2, _E_CC_P - _E_CC))
    pad_dd = jnp.broadcast_to(
        jnp.array([[0], [_N_DIS_P - 1]], jnp.int32), (2, _E_DD_P - _E_DD))
    ecc = jnp.concatenate([cc_edges.astype(jnp.int32), pad_cc], axis=1)
    edd = jnp.concatenate([dd_edges.astype(jnp.int32), pad_dd], axis=1)
    zeros = jnp.zeros((_CC_STRIPE,), f32)

    cntc_flat, cntd_flat = _sc_count_matrices(ecc, edd, zeros)
    cntc = cntc_flat.reshape(_N_CIR_P, _N_CIR_P)
    cntd = cntd_flat.reshape(_N_DIS_P, _N_DIS_P)

    pc = _N_CIR_P - _N_CIR
    pd = _N_DIS_P - _N_DIS
    mc = jnp.pad(cc_matrix, ((0, pc), (0, pc)))
    md = jnp.pad(dd_matrix, ((0, pd), (0, pd)))
    xc = jnp.pad(x_cir, ((0, pc), (0, 0)))
    xd = jnp.pad(x_dis, ((0, pd), (0, 0)))

    score, cir, dis = pl.pallas_call(
        _tc_dense_body,
        out_shape=(
            jax.ShapeDtypeStruct((_N_CIR, _N_DIS), f32),
            jax.ShapeDtypeStruct((_N_CIR, 256), f32),
            jax.ShapeDtypeStruct((_N_DIS, 256), f32),
        ),
    )(cntc, mc, xc, W_cir1, b_cir1.reshape(1, _D), W_cir2,
      b_cir2.reshape(1, _D), W_cnn_cir, b_cnn_cir.reshape(1, 256),
      cntd, md, xd, W_dis1, b_dis1.reshape(1, _D), W_dis2,
      b_dis2.reshape(1, _D), W_cnn_dis, b_cnn_dis.reshape(1, 256))

    return (score, cir, dis)


# speedup vs baseline: 1.6034x; 1.0381x over previous
import functools

import jax
import jax.numpy as jnp
from jax import lax
from jax.experimental import pallas as pl
from jax.experimental.pallas import tpu as pltpu
from jax.experimental.pallas import tpu_sc as plsc

_N_CIR = 585
_N_DIS = 88
_D = 128
_E_CC = 11700
_E_DD = 1760

_N_CIR_P = 640
_N_DIS_P = 128
_E_CC_P = 12288
_E_DD_P = 2048

_NC = 2
_NS = 16
_NW = _NC * _NS
_CC_ROWS = _N_CIR_P // _NW
_DD_ROWS = _N_DIS_P // _NW


_CC_PER_TILE = _E_CC_P // _NS
_DD_PER_TILE = _E_DD_P // _NS
_CC_STRIPE = _N_CIR_P * _N_CIR_P // _NS
_DD_STRIPE = _N_DIS_P * _N_DIS_P // _NS


def _sc_count_matrices(icc, idd, zeros, ones):
    mesh = plsc.VectorSubcoreMesh(core_axis_name="c", subcore_axis_name="s")

    @functools.partial(
        pl.kernel,
        out_type=(
            jax.ShapeDtypeStruct((_N_CIR_P * _N_CIR_P,), jnp.float32),
            jax.ShapeDtypeStruct((_N_DIS_P * _N_DIS_P,), jnp.float32),
        ),
        mesh=mesh,
        compiler_params=pltpu.CompilerParams(needs_layout_passes=False),
        scratch_types=[
            pltpu.VMEM((_CC_PER_TILE // 128, 128), jnp.int32),
            pltpu.VMEM((128,), jnp.float32),
            pltpu.VMEM_SHARED((_N_CIR_P * _N_CIR_P,), jnp.float32),
        ],
    )
    def k(icc_hbm, idd_hbm, zeros_hbm, ones_hbm, outc_hbm, outd_hbm,
          idx_v, ones_v, shr):
        core = lax.axis_index("c")
        s = lax.axis_index("s")
        pltpu.sync_copy(ones_hbm, ones_v)

        def graph(i_hbm, out_hbm, n_chunk, stripe):
            pltpu.sync_copy(i_hbm.at[s], idx_v.at[pl.ds(0, n_chunk)])
            pltpu.sync_copy(zeros_hbm.at[pl.ds(0, stripe)],
                            shr.at[pl.ds(s * stripe, stripe)])
            plsc.subcore_barrier()
            for j in range(n_chunk):
                pltpu.sync_copy(ones_v, shr.at[idx_v.at[j]], add=True)
            plsc.subcore_barrier()
            pltpu.sync_copy(shr.at[pl.ds(s * stripe, stripe)],
                            out_hbm.at[pl.ds(s * stripe, stripe)])

        @pl.when(core == 0)
        def _():
            graph(icc_hbm, outc_hbm, _CC_PER_TILE // 128, _CC_STRIPE)

        @pl.when(core == 1)
        def _():
            graph(idd_hbm, outd_hbm, _DD_PER_TILE // 128, _DD_STRIPE)

    return k(icc, idd, zeros, ones)


def _tc_dense_body(cntc_ref, mc_ref, xc_ref, wc1_ref, bc1_ref, wc2_ref,
                   bc2_ref, wcnnc_ref, bcc_ref,
                   cntd_ref, md_ref, xd_ref, wd1_ref, bd1_ref, wd2_ref,
                   bd2_ref, wcnnd_ref, bdc_ref,
                   score_ref, cir_ref, dis_ref):
    f32 = jnp.float32

    def dg(a, b, dims, prec=lax.Precision.DEFAULT):
        return lax.dot_general(a, b, (dims, ((), ())),
                               preferred_element_type=f32, precision=prec)

    def side(cnt, m, x, w1, b1, w2, b2, wcnn, bc, n):
        B = cnt * m
        ones = jnp.ones((n, 1), f32)
        deg = 1.0 + dg(B, ones, ((0,), (0,)), lax.Precision.HIGHEST)
        dinv = lax.rsqrt(deg)

        def gcn(xin, W, b):
            h = dg(xin, W, ((1,), (0,)))
            t = dinv * h
            u = dg(B, t, ((0,), (0,)), lax.Precision.HIGHEST)
            return jnp.maximum(dinv * u + (dinv * dinv) * h + b, 0.0)

        f1 = gcn(x, w1, b1)
        f2 = gcn(f1, w2, b2)
        return (dg(f1, wcnn[:, 0, :], ((1,), (1,)))
                + dg(f2, wcnn[:, 1, :], ((1,), (1,))) + bc)

    cir = side(cntc_ref[...], mc_ref[...], xc_ref[...], wc1_ref[...],
               bc1_ref[...], wc2_ref[...], bc2_ref[...], wcnnc_ref[...],
               bcc_ref[...], _N_CIR_P)
    dis = side(cntd_ref[...], md_ref[...], xd_ref[...], wd1_ref[...],
               bd1_ref[...], wd2_ref[...], bd2_ref[...], wcnnd_ref[...],
               bdc_ref[...], _N_DIS_P)
    score_ref[...] = dg(cir, dis, ((1,), (1,)))[:_N_CIR, :_N_DIS]
    cir_ref[...] = cir[:_N_CIR]
    dis_ref[...] = dis[:_N_DIS]


def kernel(x_cir, x_dis, cc_matrix, cc_edges, dd_matrix, dd_edges,
           W_cir1, b_cir1, W_cir2, b_cir2, W_dis1, b_dis1, W_dis2, b_dis2,
           W_cnn_cir, b_cnn_cir, W_cnn_dis, b_cnn_dis):
    f32 = jnp.float32

    icc = jnp.concatenate([
        cc_edges[0] * _N_CIR_P + cc_edges[1],
        jnp.full((_E_CC_P - _E_CC,), _N_CIR_P - 1, jnp.int32),
    ]).reshape(_NS, _CC_PER_TILE // 128, 128)
    idd = jnp.concatenate([
        dd_edges[0] * _N_DIS_P + dd_edges[1],
        jnp.full((_E_DD_P - _E_DD,), _N_DIS_P - 1, jnp.int32),
    ]).reshape(_NS, _DD_PER_TILE // 128, 128)
    zeros = jnp.zeros((_CC_STRIPE,), f32)
    ones = jnp.ones((128,), f32)

    cntc_flat, cntd_flat = _sc_count_matrices(icc, idd, zeros, ones)
    cntc = cntc_flat.reshape(_N_CIR_P, _N_CIR_P)
    cntd = cntd_flat.reshape(_N_DIS_P, _N_DIS_P)

    pc = _N_CIR_P - _N_CIR
    pd = _N_DIS_P - _N_DIS
    mc = jnp.pad(cc_matrix, ((0, pc), (0, pc)))
    md = jnp.pad(dd_matrix, ((0, pd), (0, pd)))
    xc = jnp.pad(x_cir, ((0, pc), (0, 0)))
    xd = jnp.pad(x_dis, ((0, pd), (0, 0)))

    score, cir, dis = pl.pallas_call(
        _tc_dense_body,
        out_shape=(
            jax.ShapeDtypeStruct((_N_CIR, _N_DIS), f32),
            jax.ShapeDtypeStruct((_N_CIR, 256), f32),
            jax.ShapeDtypeStruct((_N_DIS, 256), f32),
        ),
    )(cntc, mc, xc, W_cir1, b_cir1.reshape(1, _D), W_cir2,
      b_cir2.reshape(1, _D), W_cnn_cir, b_cnn_cir.reshape(1, 256),
      cntd, md, xd, W_dis1, b_dis1.reshape(1, _D), W_dis2,
      b_dis2.reshape(1, _D), W_cnn_dis, b_cnn_dis.reshape(1, 256))

    return (score, cir, dis)


# speedup vs baseline: 1.6575x; 1.0337x over previous
import functools

import jax
import jax.numpy as jnp
from jax import lax
from jax.experimental import pallas as pl
from jax.experimental.pallas import tpu as pltpu
from jax.experimental.pallas import tpu_sc as plsc

_N_CIR = 585
_N_DIS = 88
_D = 128
_E_CC = 11700
_E_DD = 1760

_N_CIR_P = 640
_N_DIS_P = 128
_E_CC_P = 12288
_E_DD_P = 2048

_NC = 2
_NS = 16
_NW = _NC * _NS
_CC_ROWS = _N_CIR_P // _NW
_DD_ROWS = _N_DIS_P // _NW


_CC_PER_TILE = _E_CC_P // _NS
_DD_PER_TILE = _E_DD_P // _NS
_CC_STRIPE = _N_CIR_P * _N_CIR_P // _NS
_DD_STRIPE = _N_DIS_P * _N_DIS_P // _NS


def _sc_count_matrices(icc, idd, zeros, ones):
    mesh = plsc.VectorSubcoreMesh(core_axis_name="c", subcore_axis_name="s")

    @functools.partial(
        pl.kernel,
        out_type=(
            jax.ShapeDtypeStruct((_N_CIR_P, _N_CIR_P), jnp.float32),
            jax.ShapeDtypeStruct((_N_DIS_P, _N_DIS_P), jnp.float32),
        ),
        mesh=mesh,
        compiler_params=pltpu.CompilerParams(needs_layout_passes=False),
        scratch_types=[
            pltpu.VMEM((_CC_PER_TILE // 128, 128), jnp.int32),
            pltpu.VMEM((128,), jnp.float32),
            pltpu.VMEM_SHARED((_N_CIR_P * _N_CIR_P,), jnp.float32),
            pltpu.SemaphoreType.DMA,
        ],
    )
    def k(icc_hbm, idd_hbm, zeros_hbm, ones_hbm, outc_hbm, outd_hbm,
          idx_v, ones_v, shr, sem):
        core = lax.axis_index("c")
        s = lax.axis_index("s")
        pltpu.sync_copy(ones_hbm, ones_v)

        def graph(i_hbm, out_hbm, n_chunk, stripe, npad):
            rows = stripe // npad
            base = s * rows
            pltpu.sync_copy(i_hbm.at[s], idx_v.at[pl.ds(0, n_chunk)])
            pltpu.sync_copy(zeros_hbm.at[pl.ds(0, stripe)],
                            shr.at[pl.ds(s * stripe, stripe)])
            plsc.subcore_barrier()
            for j in range(n_chunk):
                pltpu.sync_copy(ones_v, shr.at[idx_v.at[j]], add=True)
            plsc.subcore_barrier()
            copies = [
                pltpu.async_copy(
                    shr.at[pl.ds((base + j) * npad, npad)],
                    out_hbm.at[base + j], sem)
                for j in range(rows)
            ]
            for cp in copies:
                cp.wait()

        @pl.when(core == 0)
        def _():
            graph(icc_hbm, outc_hbm, _CC_PER_TILE // 128, _CC_STRIPE, _N_CIR_P)

        @pl.when(core == 1)
        def _():
            graph(idd_hbm, outd_hbm, _DD_PER_TILE // 128, _DD_STRIPE, _N_DIS_P)

    return k(icc, idd, zeros, ones)


def _tc_dense_body(cntc_ref, mc_ref, xc_ref, wc1_ref, bc1_ref, wc2_ref,
                   bc2_ref, wcnnc_ref, bcc_ref,
                   cntd_ref, md_ref, xd_ref, wd1_ref, bd1_ref, wd2_ref,
                   bd2_ref, wcnnd_ref, bdc_ref,
                   score_ref, cir_ref, dis_ref):
    f32 = jnp.float32

    def dg(a, b, dims, prec=lax.Precision.DEFAULT):
        return lax.dot_general(a, b, (dims, ((), ())),
                               preferred_element_type=f32, precision=prec)

    def side(cnt, m, x, w1, b1, w2, b2, wcnn, bc, n):
        B = cnt * m
        ones = jnp.ones((n, 1), f32)
        deg = 1.0 + dg(B, ones, ((0,), (0,)), lax.Precision.HIGHEST)
        dinv = lax.rsqrt(deg)

        def gcn(xin, W, b):
            h = dg(xin, W, ((1,), (0,)))
            t = dinv * h
            u = dg(B, t, ((0,), (0,)), lax.Precision.HIGHEST)
            return jnp.maximum(dinv * u + (dinv * dinv) * h + b, 0.0)

        f1 = gcn(x, w1, b1)
        f2 = gcn(f1, w2, b2)
        return (dg(f1, wcnn[:, 0, :], ((1,), (1,)))
                + dg(f2, wcnn[:, 1, :], ((1,), (1,))) + bc)

    cir = side(cntc_ref[...], mc_ref[...], xc_ref[...], wc1_ref[...],
               bc1_ref[...], wc2_ref[...], bc2_ref[...], wcnnc_ref[...],
               bcc_ref[...], _N_CIR_P)
    dis = side(cntd_ref[...], md_ref[...], xd_ref[...], wd1_ref[...],
               bd1_ref[...], wd2_ref[...], bd2_ref[...], wcnnd_ref[...],
               bdc_ref[...], _N_DIS_P)
    score_ref[...] = dg(cir, dis, ((1,), (1,)))[:_N_CIR, :_N_DIS]
    cir_ref[...] = cir[:_N_CIR]
    dis_ref[...] = dis[:_N_DIS]


def kernel(x_cir, x_dis, cc_matrix, cc_edges, dd_matrix, dd_edges,
           W_cir1, b_cir1, W_cir2, b_cir2, W_dis1, b_dis1, W_dis2, b_dis2,
           W_cnn_cir, b_cnn_cir, W_cnn_dis, b_cnn_dis):
    f32 = jnp.float32

    icc = jnp.concatenate([
        cc_edges[0] * _N_CIR_P + cc_edges[1],
        jnp.full((_E_CC_P - _E_CC,), _N_CIR_P - 1, jnp.int32),
    ]).reshape(_NS, _CC_PER_TILE // 128, 128)
    idd = jnp.concatenate([
        dd_edges[0] * _N_DIS_P + dd_edges[1],
        jnp.full((_E_DD_P - _E_DD,), _N_DIS_P - 1, jnp.int32),
    ]).reshape(_NS, _DD_PER_TILE // 128, 128)
    zeros = jnp.zeros((_CC_STRIPE,), f32)
    ones = jnp.ones((128,), f32)

    cntc, cntd = _sc_count_matrices(icc, idd, zeros, ones)

    pc = _N_CIR_P - _N_CIR
    pd = _N_DIS_P - _N_DIS
    mc = jnp.pad(cc_matrix, ((0, pc), (0, pc)))
    md = jnp.pad(dd_matrix, ((0, pd), (0, pd)))
    xc = jnp.pad(x_cir, ((0, pc), (0, 0)))
    xd = jnp.pad(x_dis, ((0, pd), (0, 0)))

    score, cir, dis = pl.pallas_call(
        _tc_dense_body,
        out_shape=(
            jax.ShapeDtypeStruct((_N_CIR, _N_DIS), f32),
            jax.ShapeDtypeStruct((_N_CIR, 256), f32),
            jax.ShapeDtypeStruct((_N_DIS, 256), f32),
        ),
    )(cntc, mc, xc, W_cir1, b_cir1.reshape(1, _D), W_cir2,
      b_cir2.reshape(1, _D), W_cnn_cir, b_cnn_cir.reshape(1, 256),
      cntd, md, xd, W_dis1, b_dis1.reshape(1, _D), W_dis2,
      b_dis2.reshape(1, _D), W_cnn_dis, b_cnn_dis.reshape(1, 256))

    return (score, cir, dis)


# speedup vs baseline: 1.6682x; 1.0065x over previous
import functools

import jax
import jax.numpy as jnp
from jax import lax
from jax.experimental import pallas as pl
from jax.experimental.pallas import tpu as pltpu
from jax.experimental.pallas import tpu_sc as plsc

_N_CIR = 585
_N_DIS = 88
_D = 128
_E_CC = 11700
_E_DD = 1760

_N_CIR_P = 640
_N_DIS_P = 128
_E_CC_P = 12288
_E_DD_P = 2048

_NC = 2
_NS = 16
_NW = _NC * _NS
_CC_ROWS = _N_CIR_P // _NW
_DD_ROWS = _N_DIS_P // _NW


_CC_PER_TILE = _E_CC_P // _NS
_DD_PER_TILE = _E_DD_P // _NS
_CC_STRIPE = _N_CIR_P * _N_CIR_P // _NS
_DD_STRIPE = _N_DIS_P * _N_DIS_P // _NS


def _sc_count_matrices(icc, idd, zeros, ones):
    mesh = plsc.VectorSubcoreMesh(core_axis_name="c", subcore_axis_name="s")

    @functools.partial(
        pl.kernel,
        out_type=(
            jax.ShapeDtypeStruct((_N_CIR_P, _N_CIR_P), jnp.float32),
            jax.ShapeDtypeStruct((_N_DIS_P, _N_DIS_P), jnp.float32),
        ),
        mesh=mesh,
        compiler_params=pltpu.CompilerParams(needs_layout_passes=False),
        scratch_types=[
            pltpu.VMEM((_CC_PER_TILE // 128, 128), jnp.int32),
            pltpu.VMEM((128,), jnp.float32),
            pltpu.VMEM_SHARED((_N_CIR_P * _N_CIR_P,), jnp.float32),
            pltpu.SemaphoreType.DMA,
        ],
    )
    def k(icc_hbm, idd_hbm, zeros_hbm, ones_hbm, outc_hbm, outd_hbm,
          idx_v, ones_v, shr, sem):
        core = lax.axis_index("c")
        s = lax.axis_index("s")
        pltpu.sync_copy(ones_hbm, ones_v)

        def graph(i_hbm, out_hbm, n_chunk, stripe, npad):
            rows = stripe // npad
            base = s * rows
            pltpu.sync_copy(i_hbm.at[s], idx_v.at[pl.ds(0, n_chunk)])
            pltpu.sync_copy(zeros_hbm.at[pl.ds(0, stripe)],
                            shr.at[pl.ds(s * stripe, stripe)])
            plsc.subcore_barrier()
            for j in range(n_chunk):
                pltpu.sync_copy(ones_v, shr.at[idx_v.at[j]], add=True)
            plsc.subcore_barrier()
            copies = [
                pltpu.async_copy(
                    shr.at[pl.ds((base + j) * npad, npad)],
                    out_hbm.at[base + j], sem)
                for j in range(rows)
            ]
            for cp in copies:
                cp.wait()

        @pl.when(core == 0)
        def _():
            graph(icc_hbm, outc_hbm, _CC_PER_TILE // 128, _CC_STRIPE, _N_CIR_P)

        @pl.when(core == 1)
        def _():
            graph(idd_hbm, outd_hbm, _DD_PER_TILE // 128, _DD_STRIPE, _N_DIS_P)

    return k(icc, idd, zeros, ones)


def _tc_dense_body(cntc_ref, mc_ref, xc_ref, wc1_ref, bc1_ref, wc2_ref,
                   bc2_ref, wcnnc_ref, bcc_ref,
                   cntd_ref, md_ref, xd_ref, wd1_ref, bd1_ref, wd2_ref,
                   bd2_ref, wcnnd_ref, bdc_ref,
                   score_ref, cir_ref, dis_ref,
                   cntc_v, mc_v, sem1, sem2):
    f32 = jnp.float32
    cp1 = pltpu.make_async_copy(cntc_ref, cntc_v, sem1)
    cp2 = pltpu.make_async_copy(mc_ref, mc_v, sem2)
    cp1.start()
    cp2.start()

    def dg(a, b, dims, prec=lax.Precision.DEFAULT):
        return lax.dot_general(a, b, (dims, ((), ())),
                               preferred_element_type=f32, precision=prec)

    def side(B, x, w1, b1, w2, b2, wcnn, bc, n, h1=None):
        ones = jnp.ones((n, 1), f32)
        deg = 1.0 + dg(B, ones, ((0,), (0,)), lax.Precision.HIGHEST)
        dinv = lax.rsqrt(deg)

        def gcn(xin, W, b, h=None):
            if h is None:
                h = dg(xin, W, ((1,), (0,)))
            t = dinv * h
            u = dg(B, t, ((0,), (0,)), lax.Precision.HIGHEST)
            return jnp.maximum(dinv * u + (dinv * dinv) * h + b, 0.0)

        f1 = gcn(x, w1, b1, h1)
        f2 = gcn(f1, w2, b2)
        return (dg(f1, wcnn[:, 0, :], ((1,), (1,)))
                + dg(f2, wcnn[:, 1, :], ((1,), (1,))) + bc)

    dis = side(cntd_ref[...] * md_ref[...], xd_ref[...], wd1_ref[...],
               bd1_ref[...], wd2_ref[...], bd2_ref[...], wcnnd_ref[...],
               bdc_ref[...], _N_DIS_P)
    h1c = dg(xc_ref[...], wc1_ref[...], ((1,), (0,)))
    cp1.wait()
    cp2.wait()
    cir = side(cntc_v[...] * mc_v[...], xc_ref[...], wc1_ref[...],
               bc1_ref[...], wc2_ref[...], bc2_ref[...], wcnnc_ref[...],
               bcc_ref[...], _N_CIR_P, h1=h1c)
    score_ref[...] = dg(cir, dis, ((1,), (1,)))[:_N_CIR, :_N_DIS]
    cir_ref[...] = cir[:_N_CIR]
    dis_ref[...] = dis[:_N_DIS]


def kernel(x_cir, x_dis, cc_matrix, cc_edges, dd_matrix, dd_edges,
           W_cir1, b_cir1, W_cir2, b_cir2, W_dis1, b_dis1, W_dis2, b_dis2,
           W_cnn_cir, b_cnn_cir, W_cnn_dis, b_cnn_dis):
    f32 = jnp.float32

    icc = jnp.concatenate([
        cc_edges[0] * _N_CIR_P + cc_edges[1],
        jnp.full((_E_CC_P - _E_CC,), _N_CIR_P - 1, jnp.int32),
    ]).reshape(_NS, _CC_PER_TILE // 128, 128)
    idd = jnp.concatenate([
        dd_edges[0] * _N_DIS_P + dd_edges[1],
        jnp.full((_E_DD_P - _E_DD,), _N_DIS_P - 1, jnp.int32),
    ]).reshape(_NS, _DD_PER_TILE // 128, 128)
    zeros = jnp.zeros((_CC_STRIPE,), f32)
    ones = jnp.ones((128,), f32)

    cntc, cntd = _sc_count_matrices(icc, idd, zeros, ones)

    pc = _N_CIR_P - _N_CIR
    pd = _N_DIS_P - _N_DIS
    mc = jnp.pad(cc_matrix, ((0, pc), (0, pc)))
    md = jnp.pad(dd_matrix, ((0, pd), (0, pd)))
    xc = jnp.pad(x_cir, ((0, pc), (0, 0)))
    xd = jnp.pad(x_dis, ((0, pd), (0, 0)))

    any_spec = pl.BlockSpec(memory_space=pl.ANY)
    vmem_spec = pl.BlockSpec(memory_space=pltpu.MemorySpace.VMEM)
    score, cir, dis = pl.pallas_call(
        _tc_dense_body,
        in_specs=[any_spec, any_spec] + [vmem_spec] * 16,
        scratch_shapes=[
            pltpu.VMEM((_N_CIR_P, _N_CIR_P), f32),
            pltpu.VMEM((_N_CIR_P, _N_CIR_P), f32),
            pltpu.SemaphoreType.DMA,
            pltpu.SemaphoreType.DMA,
        ],
        out_shape=(
            jax.ShapeDtypeStruct((_N_CIR, _N_DIS), f32),
            jax.ShapeDtypeStruct((_N_CIR, 256), f32),
            jax.ShapeDtypeStruct((_N_DIS, 256), f32),
        ),
    )(cntc, mc, xc, W_cir1, b_cir1.reshape(1, _D), W_cir2,
      b_cir2.reshape(1, _D), W_cnn_cir, b_cnn_cir.reshape(1, 256),
      cntd, md, xd, W_dis1, b_dis1.reshape(1, _D), W_dis2,
      b_dis2.reshape(1, _D), W_cnn_dis, b_cnn_dis.reshape(1, 256))

    return (score, cir, dis)


# speedup vs baseline: 1.7085x; 1.0242x over previous
import functools

import jax
import jax.numpy as jnp
from jax import lax
from jax.experimental import pallas as pl
from jax.experimental.pallas import tpu as pltpu
from jax.experimental.pallas import tpu_sc as plsc

_N_CIR = 585
_N_DIS = 88
_D = 128
_E_CC = 11700
_E_DD = 1760

_N_CIR_P = 640
_N_DIS_P = 128
_E_CC_P = 12288
_E_DD_P = 2048

_NC = 2
_NS = 16
_NW = _NC * _NS
_CC_ROWS = _N_CIR_P // _NW
_DD_ROWS = _N_DIS_P // _NW


_CC_PER_TILE = _E_CC_P // _NS
_DD_PER_TILE = _E_DD_P // _NS
_CC_STRIPE = _N_CIR_P * _N_CIR_P // _NS
_DD_STRIPE = _N_DIS_P * _N_DIS_P // _NS


def _sc_count_matrices(icc, idd, zeros, ones):
    mesh = plsc.VectorSubcoreMesh(core_axis_name="c", subcore_axis_name="s")

    @functools.partial(
        pl.kernel,
        out_type=(
            jax.ShapeDtypeStruct((_N_CIR_P, _N_CIR_P), jnp.float32),
            jax.ShapeDtypeStruct((_N_DIS_P, _N_DIS_P), jnp.float32),
        ),
        mesh=mesh,
        compiler_params=pltpu.CompilerParams(needs_layout_passes=False),
        scratch_types=[
            pltpu.VMEM((_CC_PER_TILE // 128, 128), jnp.int32),
            pltpu.VMEM((128,), jnp.float32),
            pltpu.VMEM_SHARED((_N_CIR_P * _N_CIR_P,), jnp.float32),
            pltpu.SemaphoreType.DMA,
        ],
    )
    def k(icc_hbm, idd_hbm, zeros_hbm, ones_hbm, outc_hbm, outd_hbm,
          idx_v, ones_v, shr, sem):
        core = lax.axis_index("c")
        s = lax.axis_index("s")
        pltpu.sync_copy(ones_hbm, ones_v)

        def graph(i_hbm, out_hbm, n_chunk, stripe, npad):
            rows = stripe // npad
            base = s * rows
            cp_i = pltpu.make_async_copy(
                i_hbm.at[s], idx_v.at[pl.ds(0, n_chunk)], sem)
            cp_z = pltpu.make_async_copy(
                zeros_hbm.at[pl.ds(0, stripe)],
                shr.at[pl.ds(s * stripe, stripe)], sem)
            cp_i.start()
            cp_z.start()
            cp_i.wait()
            cp_z.wait()
            plsc.subcore_barrier()
            adds = [
                pltpu.make_async_copy(ones_v, shr.at[idx_v.at[j]], sem)
                for j in range(n_chunk)
            ]
            for cp in adds:
                cp.start(add=True)
            for cp in adds:
                cp.wait()
            plsc.subcore_barrier()
            copies = [
                pltpu.async_copy(
                    shr.at[pl.ds((base + j) * npad, npad)],
                    out_hbm.at[base + j], sem)
                for j in range(rows)
            ]
            for cp in copies:
                cp.wait()

        @pl.when(core == 0)
        def _():
            graph(icc_hbm, outc_hbm, _CC_PER_TILE // 128, _CC_STRIPE, _N_CIR_P)

        @pl.when(core == 1)
        def _():
            graph(idd_hbm, outd_hbm, _DD_PER_TILE // 128, _DD_STRIPE, _N_DIS_P)

    return k(icc, idd, zeros, ones)


def _tc_dense_body(cntc_ref, mc_ref, xc_ref, wc1_ref, bc1_ref, wc2_ref,
                   bc2_ref, wcnnc_ref, bcc_ref,
                   cntd_ref, md_ref, xd_ref, wd1_ref, bd1_ref, wd2_ref,
                   bd2_ref, wcnnd_ref, bdc_ref,
                   score_ref, cir_ref, dis_ref,
                   cntc_v, mc_v, sem1, sem2):
    f32 = jnp.float32
    cp1 = pltpu.make_async_copy(cntc_ref, cntc_v, sem1)
    cp2 = pltpu.make_async_copy(mc_ref, mc_v, sem2)
    cp1.start()
    cp2.start()

    def dg(a, b, dims, prec=lax.Precision.DEFAULT):
        return lax.dot_general(a, b, (dims, ((), ())),
                               preferred_element_type=f32, precision=prec)

    def side(B, x, w1, b1, w2, b2, wcnn, bc, n, h1=None):
        ones = jnp.ones((n, 1), f32)
        deg = 1.0 + dg(B, ones, ((0,), (0,)), lax.Precision.HIGHEST)
        dinv = lax.rsqrt(deg)

        def gcn(xin, W, b, h=None):
            if h is None:
                h = dg(xin, W, ((1,), (0,)))
            t = dinv * h
            u = dg(B, t, ((0,), (0,)), lax.Precision.HIGHEST)
            return jnp.maximum(dinv * u + (dinv * dinv) * h + b, 0.0)

        f1 = gcn(x, w1, b1, h1)
        f2 = gcn(f1, w2, b2)
        return (dg(f1, wcnn[:, 0, :], ((1,), (1,)))
                + dg(f2, wcnn[:, 1, :], ((1,), (1,))) + bc)

    dis = side(cntd_ref[...] * md_ref[...], xd_ref[...], wd1_ref[...],
               bd1_ref[...], wd2_ref[...], bd2_ref[...], wcnnd_ref[...],
               bdc_ref[...], _N_DIS_P)
    h1c = dg(xc_ref[...], wc1_ref[...], ((1,), (0,)))
    cp1.wait()
    cp2.wait()
    cir = side(cntc_v[...] * mc_v[...], xc_ref[...], wc1_ref[...],
               bc1_ref[...], wc2_ref[...], bc2_ref[...], wcnnc_ref[...],
               bcc_ref[...], _N_CIR_P, h1=h1c)
    score_ref[...] = dg(cir, dis, ((1,), (1,)))[:_N_CIR, :_N_DIS]
    cir_ref[...] = cir[:_N_CIR]
    dis_ref[...] = dis[:_N_DIS]


def kernel(x_cir, x_dis, cc_matrix, cc_edges, dd_matrix, dd_edges,
           W_cir1, b_cir1, W_cir2, b_cir2, W_dis1, b_dis1, W_dis2, b_dis2,
           W_cnn_cir, b_cnn_cir, W_cnn_dis, b_cnn_dis):
    f32 = jnp.float32

    icc = jnp.concatenate([
        cc_edges[0] * _N_CIR_P + cc_edges[1],
        jnp.full((_E_CC_P - _E_CC,), _N_CIR_P - 1, jnp.int32),
    ]).reshape(_NS, _CC_PER_TILE // 128, 128)
    idd = jnp.concatenate([
        dd_edges[0] * _N_DIS_P + dd_edges[1],
        jnp.full((_E_DD_P - _E_DD,), _N_DIS_P - 1, jnp.int32),
    ]).reshape(_NS, _DD_PER_TILE // 128, 128)
    zeros = jnp.zeros((_CC_STRIPE,), f32)
    ones = jnp.ones((128,), f32)

    cntc, cntd = _sc_count_matrices(icc, idd, zeros, ones)

    pc = _N_CIR_P - _N_CIR
    pd = _N_DIS_P - _N_DIS
    mc = jnp.pad(cc_matrix, ((0, pc), (0, pc)))
    md = jnp.pad(dd_matrix, ((0, pd), (0, pd)))
    xc = jnp.pad(x_cir, ((0, pc), (0, 0)))
    xd = jnp.pad(x_dis, ((0, pd), (0, 0)))

    any_spec = pl.BlockSpec(memory_space=pl.ANY)
    vmem_spec = pl.BlockSpec(memory_space=pltpu.MemorySpace.VMEM)
    score, cir, dis = pl.pallas_call(
        _tc_dense_body,
        in_specs=[any_spec, any_spec] + [vmem_spec] * 16,
        scratch_shapes=[
            pltpu.VMEM((_N_CIR_P, _N_CIR_P), f32),
            pltpu.VMEM((_N_CIR_P, _N_CIR_P), f32),
            pltpu.SemaphoreType.DMA,
            pltpu.SemaphoreType.DMA,
        ],
        out_shape=(
            jax.ShapeDtypeStruct((_N_CIR, _N_DIS), f32),
            jax.ShapeDtypeStruct((_N_CIR, 256), f32),
            jax.ShapeDtypeStruct((_N_DIS, 256), f32),
        ),
    )(cntc, mc, xc, W_cir1, b_cir1.reshape(1, _D), W_cir2,
      b_cir2.reshape(1, _D), W_cnn_cir, b_cnn_cir.reshape(1, 256),
      cntd, md, xd, W_dis1, b_dis1.reshape(1, _D), W_dis2,
      b_dis2.reshape(1, _D), W_cnn_dis, b_cnn_dis.reshape(1, 256))

    return (score, cir, dis)
